# Initial kernel scaffold; baseline (speedup 1.0000x reference)
import jax
import jax.numpy as jnp
from jax.experimental import pallas as pl


def kernel(x1, x2, comp_features, edge_index1, edge_index2, batch1, batch2, W1, b1, W2, b2, W3, b3, Wc, bc, Wf1, bf1, Wf2, bf2, Wf3, bf3):
    raise NotImplementedError("write your pallas kernel here")


# speedup vs baseline: 17.5461x; 17.5461x over previous
import functools

import jax
import jax.numpy as jnp
from jax import lax
from jax.experimental import pallas as pl
from jax.experimental.pallas import tpu as pltpu
from jax.experimental.pallas import tpu_sc as plsc

N = 10000
NG = 64
NGT = 2 * NG
DIN = 128
DH = 64
DE = 32
E = 320000

NPAD = 10240
NT = 2 * NPAD
NSUB = 16
RSUB = NT // NSUB
DUMMY = 10200
DGW = 16
C = 128
NW = 32
ETOT = 2 * E
NCH = -(-ETOT // (NW * C))
EP = NW * C * NCH

RB = 1280
GRID = NT // RB

_SC_PARAMS = pltpu.CompilerParams(use_tc_tiling_on_sc=False)


def _sc_mesh():
    return plsc.VectorSubcoreMesh(core_axis_name="c", subcore_axis_name="s",
                                  num_cores=2, num_subcores=NSUB)

_HI = jax.lax.Precision.HIGHEST


def _mm(a, b):
    return jax.lax.dot_general(a, b, (((1,), (0,)), ((), ())),
                               precision=_HI,
                               preferred_element_type=jnp.float32)


def _deg_call(dst_w, zeros_nt, ones_c):
    @functools.partial(
        pl.kernel,
        out_type=jax.ShapeDtypeStruct((2, NT, DGW), jnp.float32),
        mesh=_sc_mesh(),
        compiler_params=_SC_PARAMS,
        scratch_types=[
            pltpu.VMEM((NCH, C), jnp.int32),
            pltpu.VMEM((C, DGW), jnp.float32),
            pltpu.VMEM_SHARED((NT, DGW), jnp.float32),
        ],
    )
    def deg_kernel(dst_hbm, zero_hbm, one_hbm, out_hbm, idx_v, ones_v, acc_sh):
        cid = lax.axis_index("c")
        sid = lax.axis_index("s")
        wid = cid * NSUB + sid
        r0 = sid * RSUB
        pltpu.sync_copy(zero_hbm.at[pl.ds(r0, RSUB)], acc_sh.at[pl.ds(r0, RSUB)])
        pltpu.sync_copy(one_hbm, ones_v)
        pltpu.sync_copy(dst_hbm.at[wid], idx_v)
        plsc.subcore_barrier()

        @pl.loop(0, NCH)
        def _(j):
            pltpu.sync_copy(ones_v, acc_sh.at[idx_v.at[j]], add=True)

        plsc.subcore_barrier()
        pltpu.sync_copy(acc_sh.at[pl.ds(r0, RSUB)],
                        out_hbm.at[cid, pl.ds(r0, RSUB)])

    return deg_kernel(dst_w, zeros_nt, ones_c)


def _edge_call(hp, src_w, dst_w, d):
    @functools.partial(
        pl.kernel,
        out_type=jax.ShapeDtypeStruct((2, NT, d), jnp.float32),
        mesh=_sc_mesh(),
        compiler_params=_SC_PARAMS,
        scratch_types=[
            pltpu.VMEM((NCH, C), jnp.int32),
            pltpu.VMEM((NCH, C), jnp.int32),
            pltpu.VMEM((C, d), jnp.float32),
            pltpu.VMEM_SHARED((NT, d), jnp.float32),
        ],
    )
    def edge_kernel(hp_hbm, src_hbm, dst_hbm, out_hbm,
                    src_v, dst_v, buf_v, acc_sh):
        cid = lax.axis_index("c")
        sid = lax.axis_index("s")
        wid = cid * NSUB + sid
        r0 = sid * RSUB
        pltpu.sync_copy(hp_hbm.at[pl.ds(r0, RSUB)], acc_sh.at[pl.ds(r0, RSUB)])
        pltpu.sync_copy(src_hbm.at[wid], src_v)
        pltpu.sync_copy(dst_hbm.at[wid], dst_v)
        plsc.subcore_barrier()

        @pl.loop(0, NCH)
        def _(j):
            pltpu.sync_copy(hp_hbm.at[src_v.at[j]], buf_v)
            pltpu.sync_copy(buf_v, acc_sh.at[dst_v.at[j]], add=True)

        plsc.subcore_barrier()
        pltpu.sync_copy(acc_sh.at[pl.ds(r0, RSUB)],
                        out_hbm.at[cid, pl.ds(r0, RSUB)])

    return edge_kernel(hp, src_w, dst_w)


def _k1_body(x_ref, w_ref, degp_ref, hp_ref, dinv_ref):
    deg = degp_ref[0] + degp_ref[1] + 1.0
    dinv = jax.lax.rsqrt(deg)
    hp_ref[...] = _mm(x_ref[...], w_ref[...]) * dinv
    dinv_ref[...] = dinv


def _first_layer(xs, w1, degp):
    return pl.pallas_call(
        _k1_body,
        grid=(GRID,),
        in_specs=[
            pl.BlockSpec((RB, DIN), lambda i: (i, 0)),
            pl.BlockSpec((DIN, DH), lambda i: (0, 0)),
            pl.BlockSpec((2, RB, 1), lambda i: (0, i, 0)),
        ],
        out_specs=[
            pl.BlockSpec((RB, DH), lambda i: (i, 0)),
            pl.BlockSpec((RB, 1), lambda i: (i, 0)),
        ],
        out_shape=[
            jax.ShapeDtypeStruct((NT, DH), jnp.float32),
            jax.ShapeDtypeStruct((NT, 1), jnp.float32),
        ],
    )(xs, w1, degp)


def _mid_body(part_ref, hp_ref, dinv_ref, b_ref, w_ref, out_ref):
    dinv = dinv_ref[...]
    a = dinv * (part_ref[0] + part_ref[1] - hp_ref[...]) + b_ref[...]
    a = jnp.maximum(a, 0.0)
    out_ref[...] = _mm(a, w_ref[...]) * dinv


def _mid_layer(part, hp, dinv, b_row, w, d_in, d_out):
    return pl.pallas_call(
        _mid_body,
        grid=(GRID,),
        in_specs=[
            pl.BlockSpec((2, RB, d_in), lambda i: (0, i, 0)),
            pl.BlockSpec((RB, d_in), lambda i: (i, 0)),
            pl.BlockSpec((RB, 1), lambda i: (i, 0)),
            pl.BlockSpec((1, d_in), lambda i: (0, 0)),
            pl.BlockSpec((d_in, d_out), lambda i: (0, 0)),
        ],
        out_specs=pl.BlockSpec((RB, d_out), lambda i: (i, 0)),
        out_shape=jax.ShapeDtypeStruct((NT, d_out), jnp.float32),
    )(part, hp, dinv, b_row, w)


def _pool_body(part_ref, hp_ref, dinv_ref, b_ref, batch_ref,
               sums_ref, cnts_ref):
    i = pl.program_id(0)

    @pl.when(i == 0)
    def _():
        sums_ref[...] = jnp.zeros_like(sums_ref)
        cnts_ref[...] = jnp.zeros_like(cnts_ref)

    dinv = dinv_ref[...]
    h = dinv * (part_ref[0] + part_ref[1] - hp_ref[...]) + b_ref[...]
    gid = jax.lax.broadcasted_iota(jnp.int32, (1, NGT), 1)
    oh = (batch_ref[...] == gid).astype(jnp.float32)
    sums_ref[...] += jax.lax.dot_general(
        oh, h, (((0,), (0,)), ((), ())),
        precision=_HI, preferred_element_type=jnp.float32)
    cnts_ref[...] += jnp.sum(oh, axis=0)[:, None]


def _pool_layer(part, hp, dinv, b_row, batch_col):
    return pl.pallas_call(
        _pool_body,
        grid=(GRID,),
        in_specs=[
            pl.BlockSpec((2, RB, DE), lambda i: (0, i, 0)),
            pl.BlockSpec((RB, DE), lambda i: (i, 0)),
            pl.BlockSpec((RB, 1), lambda i: (i, 0)),
            pl.BlockSpec((1, DE), lambda i: (0, 0)),
            pl.BlockSpec((RB, 1), lambda i: (i, 0)),
        ],
        out_specs=[
            pl.BlockSpec((NGT, DE), lambda i: (0, 0)),
            pl.BlockSpec((NGT, 1), lambda i: (0, 0)),
        ],
        out_shape=[
            jax.ShapeDtypeStruct((NGT, DE), jnp.float32),
            jax.ShapeDtypeStruct((NGT, 1), jnp.float32),
        ],
    )(part, hp, dinv, b_row, batch_col)


def _head_body(sums_ref, cnts_ref, comp_ref, wc_ref, bc_ref,
               wa_ref, wb_ref, wcf_ref, bf1_ref, wf2_ref, bf2_ref,
               wf3_ref, bf3_ref, out_ref):
    emb = sums_ref[...] / jnp.maximum(cnts_ref[...], 1.0)
    e1 = emb[:NG]
    e2 = emb[NG:]
    cf = jnp.maximum(_mm(comp_ref[...], wc_ref[...]) + bc_ref[...], 0.0)
    z = _mm(e1, wa_ref[...]) + _mm(e2, wb_ref[...]) + _mm(cf, wcf_ref[...])
    z = jnp.maximum(z + bf1_ref[...], 0.0)
    z = jnp.maximum(_mm(z, wf2_ref[...]) + bf2_ref[...], 0.0)
    z = _mm(z, wf3_ref[...]) + bf3_ref[...]
    out_ref[...] = jax.nn.sigmoid(z)


def _head(sums, cnts, comp, wc, bc, wa, wb, wcf, bf1, wf2, bf2, wf3, bf3):
    return pl.pallas_call(
        _head_body,
        out_shape=jax.ShapeDtypeStruct((NG, 1), jnp.float32),
    )(sums, cnts, comp, wc, bc, wa, wb, wcf, bf1, wf2, bf2, wf3, bf3)


def kernel(x1, x2, comp_features, edge_index1, edge_index2, batch1, batch2,
           W1, b1, W2, b2, W3, b3, Wc, bc, Wf1, bf1, Wf2, bf2, Wf3, bf3):
    f32 = jnp.float32
    i32 = jnp.int32

    zrows = jnp.zeros((NPAD - N, DIN), f32)
    xs = jnp.concatenate([x1, zrows, x2, zrows])

    epad = jnp.full((EP - ETOT,), DUMMY, i32)
    src = jnp.concatenate([edge_index1[0], edge_index2[0] + NPAD, epad])
    dst = jnp.concatenate([edge_index1[1], edge_index2[1] + NPAD, epad])
    src_w = src.reshape(NW, NCH, C)
    dst_w = dst.reshape(NW, NCH, C)

    bpad = jnp.full((NPAD - N,), NGT + 7, i32)
    batch_col = jnp.concatenate(
        [batch1, bpad, batch2 + NG, bpad]).reshape(NT, 1)

    zeros_nt = jnp.zeros((NT, DGW), f32)
    ones_c = jnp.ones((C, DGW), f32)

    degp = _deg_call(dst_w, zeros_nt, ones_c)[:, :, :1]

    hp1, dinv = _first_layer(xs, W1, degp)
    p1 = _edge_call(hp1, src_w, dst_w, DH)
    hp2 = _mid_layer(p1, hp1, dinv, b1.reshape(1, DH), W2, DH, DH)
    p2 = _edge_call(hp2, src_w, dst_w, DH)
    hp3 = _mid_layer(p2, hp2, dinv, b2.reshape(1, DH), W3, DH, DE)
    p3 = _edge_call(hp3, src_w, dst_w, DE)
    sums, cnts = _pool_layer(p3, hp3, dinv, b3.reshape(1, DE), batch_col)

    return _head(sums, cnts, comp_features,
                 Wc, bc.reshape(1, 16),
                 Wf1[:DE], Wf1[DE:2 * DE], Wf1[2 * DE:],
                 bf1.reshape(1, DH), Wf2, bf2.reshape(1, 32),
                 Wf3, bf3.reshape(1, 1))
